# baseline (speedup 1.0000x reference)
import functools

import jax
import jax.numpy as jnp
from jax import lax
from jax.experimental import pallas as pl
from jax.experimental.pallas import tpu as pltpu
from jax.experimental.pallas import tpu_sc as plsc

K = 32
M = 50000
E = 800000
NTILE = 16
NW = 32
EPW = E // NW
C = 200
NCH = EPW // C
RPT = M // NTILE

assert NCH >= 3 and (NCH - 3) % 2 == 0

_mesh = plsc.VectorSubcoreMesh(core_axis_name="c", subcore_axis_name="s")


def _mul_rows(dst_v, a_v, b_v, nrows=C):
    def blk(bi, carry):
        r0 = bi * 8
        for u in range(8):
            r = r0 + u
            for h in range(K // 16):
                sl = pl.ds(h * 16, 16)
                dst_v[r, sl] = a_v[r, sl] * b_v[r, sl]
        return carry
    lax.fori_loop(0, nrows // 8, blk, 0)


def _mul_rows_scale(dst_v, a_v, b_v, s_v, nrows=C):
    def blk(bi, carry):
        d = s_v[pl.ds(bi * 16, 16)]
        for lane in range(16):
            s = d[lane]
            r = bi * 16 + lane
            for h in range(K // 16):
                sl = pl.ds(h * 16, 16)
                dst_v[r, sl] = a_v[r, sl] * b_v[r, sl] * s
        return carry
    lax.fori_loop(0, nrows // 16, blk, 0)
    rem = nrows % 16
    if rem:
        d = s_v[pl.ds(nrows - 16, 16)]
        for lane in range(16 - rem, 16):
            s = d[lane]
            r = nrows - 16 + lane
            for h in range(K // 16):
                sl = pl.ds(h * 16, 16)
                dst_v[r, sl] = a_v[r, sl] * b_v[r, sl] * s


def _edge_pass(base0, wij_hbm, w_col, tab_hbm, dirc_hbm,
               idxi_hbm, idxj_hbm, acc_sh,
               idxi_v, idxj_v, dir_v, w_v, g_v,
               sem_i, sem_j, sem_d, sem_w, sem_g,
               dir_base=None, idx_off=None):
    scaled = dirc_hbm is not None

    def fire_loads(k, s):
        base = base0 + k * C
        pltpu.async_copy(idxi_hbm.at[pl.ds(base, C)], idxi_v[s], sem_i[s])
        pltpu.async_copy(idxj_hbm.at[pl.ds(base, C)], idxj_v[s], sem_j[s])
        pltpu.async_copy(wij_hbm.at[pl.ds(base, C), pl.ds(w_col, K)],
                         w_v[s], sem_w[s])
        if scaled:
            pltpu.async_copy(dirc_hbm.at[pl.ds(dir_base + base, C)],
                             dir_v[s], sem_d[s])

    def prep_gather(k, s):
        base = base0 + k * C
        pltpu.make_async_copy(idxj_hbm.at[pl.ds(base, C)], idxj_v[s],
                              sem_j[s]).wait()
        if idx_off is not None:
            def addblk(bi, carry):
                sl = pl.ds(bi * 16, 16)
                idxj_v[s][sl] = idxj_v[s][sl] + idx_off
                return carry
            lax.fori_loop(0, C // 16, addblk, 0)
            rem = C % 16
            if rem:
                it = lax.iota(jnp.int32, 16)
                sl = pl.ds(C - 16, 16)
                idxj_v[s][sl] = idxj_v[s][sl] + jnp.where(
                    it >= 16 - rem, idx_off, 0)
        pltpu.async_copy(tab_hbm.at[idxj_v[s]], g_v[s], sem_g[s])

    def do_chunk(k, s):
        base = base0 + k * C
        pltpu.make_async_copy(idxi_hbm.at[pl.ds(base, C)], idxi_v[s],
                              sem_i[s]).wait()
        pltpu.make_async_copy(wij_hbm.at[pl.ds(base, C), pl.ds(w_col, K)],
                              w_v[s], sem_w[s]).wait()
        if scaled:
            pltpu.make_async_copy(dirc_hbm.at[pl.ds(dir_base + base, C)],
                                  dir_v[s], sem_d[s]).wait()
        pltpu.make_async_copy(tab_hbm.at[idxj_v[s]], g_v[s], sem_g[s]).wait()
        if scaled:
            _mul_rows_scale(w_v[s], w_v[s], g_v[s], dir_v[s])
        else:
            _mul_rows(w_v[s], w_v[s], g_v[s])
        pltpu.sync_copy(w_v[s], acc_sh.at[idxi_v[s]], add=True)

    fire_loads(0, 0)
    prep_gather(0, 0)
    fire_loads(1, 1)

    def body(i, carry):
        for u in range(2):
            k = 2 * i + u
            s, s2 = u, 1 - u
            prep_gather(k + 1, s2)
            do_chunk(k, s)
            fire_loads(k + 2, s)
        return carry
    lax.fori_loop(0, (NCH - 3) // 2, body, 0)

    prep_gather(NCH - 2, 1)
    do_chunk(NCH - 3, 0)
    fire_loads(NCH - 1, 0)
    prep_gather(NCH - 1, 0)
    do_chunk(NCH - 2, 1)
    do_chunk(NCH - 1, 0)


@functools.partial(
    pl.kernel,
    out_type=jax.ShapeDtypeStruct((8, M, K), jnp.float32),
    mesh=_mesh,
    compiler_params=pltpu.CompilerParams(use_tc_tiling_on_sc=False),
    scratch_types=[
        pltpu.VMEM((C,), jnp.int32), pltpu.VMEM((C,), jnp.int32),
        pltpu.VMEM((C,), jnp.int32), pltpu.VMEM((C,), jnp.int32),
        pltpu.VMEM((C,), jnp.float32), pltpu.VMEM((C,), jnp.float32),
        pltpu.VMEM((C, K), jnp.float32), pltpu.VMEM((C, K), jnp.float32),
        pltpu.VMEM((C, K), jnp.float32), pltpu.VMEM((C, K), jnp.float32),
        pltpu.VMEM_SHARED((M, K), jnp.float32),
    ] + [pltpu.SemaphoreType.DMA] * 10,
)
def _edge_sc_kernel(wij_hbm, idxi_hbm, idxj_hbm, dir3_hbm, xq_hbm, xr_hbm,
                    t3_hbm, zeros_hbm, out_hbm,
                    ii0, ii1, ij0, ij1, d0, d1, w0, w1, g0, g1, acc_sh,
                    si0, si1, sj0, sj1, sd0, sd1, sw0, sw1, sg0, sg1):
    cid = lax.axis_index("c")
    sid = lax.axis_index("s")
    wid = cid * NTILE + sid
    base0 = wid * EPW
    rows = pl.ds(sid * RPT, RPT)
    bufs = dict(idxi_v=[ii0, ii1], idxj_v=[ij0, ij1], dir_v=[d0, d1],
                w_v=[w0, w1], g_v=[g0, g1],
                sem_i=[si0, si1], sem_j=[sj0, sj1], sem_d=[sd0, sd1],
                sem_w=[sw0, sw1], sem_g=[sg0, sg1])

    def zero_acc():
        pltpu.sync_copy(zeros_hbm, acc_sh.at[rows])

    def writeout(po):
        pltpu.sync_copy(acc_sh.at[rows], out_hbm.at[po, rows])

    zero_acc()
    plsc.subcore_barrier()
    _edge_pass(base0, wij_hbm, 0, xq_hbm, None, idxi_hbm, idxj_hbm,
               acc_sh, **bufs)
    plsc.subcore_barrier()
    writeout(cid)

    def dmu_pass(p, carry):
        zero_acc()
        plsc.subcore_barrier()
        _edge_pass(base0, wij_hbm, K, xr_hbm, dir3_hbm, idxi_hbm, idxj_hbm,
                   acc_sh, dir_base=p * E, **bufs)
        _edge_pass(base0, wij_hbm, 2 * K, t3_hbm, None, idxi_hbm, idxj_hbm,
                   acc_sh, idx_off=p * M, **bufs)
        plsc.subcore_barrier()
        writeout(2 * (p + 1) + cid)
        return carry
    lax.fori_loop(0, 3, dmu_pass, 0)


_RB = 2000


def _mlp_body(emb_ref, mu_ref, w1_ref, b1_ref, w2_ref, b2_ref,
              xq_ref, xr_ref, t3_ref):
    h = jnp.dot(emb_ref[...], w1_ref[...], preferred_element_type=jnp.float32)
    h = h + b1_ref[...]
    h = h * lax.logistic(h)
    x = jnp.dot(h, w2_ref[...], preferred_element_type=jnp.float32)
    x = x + b2_ref[...]
    xq_ref[...] = x[:, 0:K]
    xr_ref[...] = x[:, K:2 * K]
    xm = x[:, 2 * K:3 * K]
    t3_ref[0] = xm * mu_ref[:, 0:K]
    t3_ref[1] = xm * mu_ref[:, K:2 * K]
    t3_ref[2] = xm * mu_ref[:, 2 * K:3 * K]


def _combine_body(emb_ref, mu_ref, s_ref, q_ref, mo_ref):
    q_ref[...] = emb_ref[...] + s_ref[0] + s_ref[1]
    dmu = jnp.concatenate(
        [s_ref[2] + s_ref[3], s_ref[4] + s_ref[5], s_ref[6] + s_ref[7]],
        axis=-1)
    mo_ref[...] = mu_ref[...] + dmu


def kernel(atomic_numbers_embedding, mu, Wij, dir_ij, pairlist, n_atoms,
           W1, b1, W2, b2):
    del n_atoms
    n, m, k = atomic_numbers_embedding.shape
    emb2d = atomic_numbers_embedding.reshape(m, k)
    wij2d = Wij.reshape(E, 3 * K)
    mu96 = mu.reshape(M, 3 * K)

    xq, xr, t3 = pl.pallas_call(
        _mlp_body,
        grid=(M // _RB,),
        in_specs=[
            pl.BlockSpec((_RB, K), lambda i: (i, 0)),
            pl.BlockSpec((_RB, 3 * K), lambda i: (i, 0)),
            pl.BlockSpec((K, K), lambda i: (0, 0)),
            pl.BlockSpec((1, K), lambda i: (0, 0)),
            pl.BlockSpec((K, 3 * K), lambda i: (0, 0)),
            pl.BlockSpec((1, 3 * K), lambda i: (0, 0)),
        ],
        out_specs=[
            pl.BlockSpec((_RB, K), lambda i: (i, 0)),
            pl.BlockSpec((_RB, K), lambda i: (i, 0)),
            pl.BlockSpec((3, _RB, K), lambda i: (0, i, 0)),
        ],
        out_shape=[
            jax.ShapeDtypeStruct((M, K), jnp.float32),
            jax.ShapeDtypeStruct((M, K), jnp.float32),
            jax.ShapeDtypeStruct((3, M, K), jnp.float32),
        ],
    )(emb2d, mu96, W1, b1.reshape(1, K), W2, b2.reshape(1, 3 * K))

    zeros = jnp.zeros((RPT, K), jnp.float32)
    dir3 = dir_ij.T.reshape(3 * E)
    idx_i = pairlist[0]
    idx_j = pairlist[1]

    slabs = _edge_sc_kernel(wij2d, idx_i, idx_j, dir3, xq, xr,
                            t3.reshape(3 * M, K), zeros)

    q2d, mo96 = pl.pallas_call(
        _combine_body,
        grid=(M // _RB,),
        in_specs=[
            pl.BlockSpec((_RB, K), lambda i: (i, 0)),
            pl.BlockSpec((_RB, 3 * K), lambda i: (i, 0)),
            pl.BlockSpec((8, _RB, K), lambda i: (0, i, 0)),
        ],
        out_specs=[
            pl.BlockSpec((_RB, K), lambda i: (i, 0)),
            pl.BlockSpec((_RB, 3 * K), lambda i: (i, 0)),
        ],
        out_shape=[
            jax.ShapeDtypeStruct((M, K), jnp.float32),
            jax.ShapeDtypeStruct((M, 3 * K), jnp.float32),
        ],
    )(emb2d, mu96, slabs)

    return (q2d.reshape(n, m, k), mo96.reshape(M, 3, K))
